# baseline (speedup 1.0000x reference)
import functools

import jax
import jax.numpy as jnp
from jax import lax
from jax.experimental import pallas as pl
from jax.experimental.pallas import tpu as pltpu
from jax.experimental.pallas import tpu_sc as plsc

N = 10000
E = 320000
D = 128
G = 64

NP = 10112
NC = 2
NS = 16
NW = NC * NS
CH = 48
ET = E + N
K_CHUNKS = 216
EP = NW * CH * K_CHUNKS
ROWS_PER_TILE = NP // NS
BN = 1264


def _mm_body(x_ref, wl_ref, wr_ref, xl_ref, xr_ref):
    x = x_ref[...]
    xl_ref[...] = jnp.dot(x, wl_ref[...], preferred_element_type=jnp.float32)
    xr_ref[...] = jnp.dot(x, wr_ref[...], preferred_element_type=jnp.float32)


def _node_transforms(x_pad, W_l, W_r):
    bn = BN
    grid = (NP // bn,)
    return pl.pallas_call(
        _mm_body,
        grid=grid,
        in_specs=[
            pl.BlockSpec((bn, D), lambda i: (i, 0)),
            pl.BlockSpec((D, D), lambda i: (0, 0)),
            pl.BlockSpec((D, D), lambda i: (0, 0)),
        ],
        out_specs=[
            pl.BlockSpec((bn, D), lambda i: (i, 0)),
            pl.BlockSpec((bn, D), lambda i: (i, 0)),
        ],
        out_shape=[
            jax.ShapeDtypeStruct((NP, D), jnp.float32),
            jax.ShapeDtypeStruct((NP, D), jnp.float32),
        ],
    )(x_pad, W_l, W_r)


def _edge_body(xl, xr, edges, atth, accout, denout,
               src0, dst0, src1, dst1, src2, dst2,
               A0, B0, A1, B1, A2, B2,
               att_v, exb0, exb1, exb2, den_st, acc_sh, den_sh,
               s0, s1, s2, t0, t1, t2):
    c = lax.axis_index("c")
    s = lax.axis_index("s")
    wid = c * NS + s
    tb = s * ROWS_PER_TILE
    zero16 = jnp.zeros((16,), jnp.float32)
    lane = lax.iota(jnp.int32, 16)
    m0 = lane == 0

    A = A0

    def zrow(i, _):
        for v in range(8):
            A[i, pl.ds(v * 16, 16)] = zero16
        return 0
    lax.fori_loop(0, CH, zrow, 0)

    for v in range(CH // 16):
        exb0[pl.ds(v * 16, 16)] = zero16
    for j in range(ROWS_PER_TILE // CH):
        pltpu.sync_copy(A, acc_sh.at[pl.ds(tb + j * CH, CH)])
    tail = ROWS_PER_TILE % CH
    if tail:
        tb2 = tb + (ROWS_PER_TILE // CH) * CH
        pltpu.sync_copy(A.at[pl.ds(0, tail)], acc_sh.at[pl.ds(tb2, tail)])
    for j in range(ROWS_PER_TILE // CH):
        pltpu.sync_copy(exb0, den_sh.at[pl.ds(tb + j * CH, CH)])
    if tail:
        tb2 = tb + (ROWS_PER_TILE // CH) * CH
        pltpu.sync_copy(exb0.at[pl.ds(0, tail)], den_sh.at[pl.ds(tb2, tail)])
    pltpu.sync_copy(atth, att_v)
    plsc.subcore_barrier()

    att8 = [att_v[pl.ds(v * 16, 16)] for v in range(8)]
    ebase = wid * (K_CHUNKS * CH)
    bufs = ((src0, dst0, A0, B0, exb0, s0, t0),
            (src1, dst1, A1, B1, exb1, s1, t1),
            (src2, dst2, A2, B2, exb2, s2, t2))

    def issue(p, j):
        sv, dv, Ab, Bb, eb, sm, st = bufs[p]
        base = ebase + j * CH
        pltpu.sync_copy(edges.at[pl.ds(base, CH)], sv)
        pltpu.sync_copy(edges.at[pl.ds(EP + base, CH)], dv)
        pltpu.async_copy(xl.at[sv], Ab, sm)
        pltpu.async_copy(xr.at[dv], Bb, sm)

    def drain_scatter(p):
        sv, dv, Ab, Bb, eb, sm, st = bufs[p]
        pltpu.make_async_copy(Ab, acc_sh.at[dv], st).wait()
        pltpu.make_async_copy(eb, den_sh.at[dv], st).wait()

    def process(p):
        sv, dv, Ab, Bb, eb, sm, st = bufs[p]
        pltpu.make_async_copy(xl.at[sv], Ab, sm).wait()
        pltpu.make_async_copy(xr.at[dv], Bb, sm).wait()

        @plsc.parallel_loop(0, CH, step=4, unroll=8)
        def ibody(t):
            edges = tuple(t + u for u in range(4))
            avss = []
            evs = []
            for i in edges:
                avs = []
                pv = zero16
                ms = []
                for v in range(8):
                    a = Ab[i, pl.ds(v * 16, 16)]
                    b = Bb[i, pl.ds(v * 16, 16)]
                    avs.append(a)
                    z = a + b
                    z = jnp.maximum(z, z * 0.2)
                    ms.append(z * att8[v])
                pv = (((ms[0] + ms[1]) + (ms[2] + ms[3]))
                      + ((ms[4] + ms[5]) + (ms[6] + ms[7])))
                avss.append(avs)
                evs.append(jnp.exp(jnp.full((16,), jnp.sum(pv),
                                            jnp.float32)))
            for n, i in enumerate(edges):
                for v in range(8):
                    Ab[i, pl.ds(v * 16, 16)] = avss[n][v] * evs[n]
                plsc.store_scatter(eb, [jnp.full((16,), i, jnp.int32)],
                                   evs[n], mask=m0)

        pltpu.async_copy(Ab, acc_sh.at[dv], st, add=True)
        pltpu.async_copy(eb, den_sh.at[dv], st, add=True)

    issue(0, 0)
    issue(1, 1)

    def ring(m, _):
        for p in range(3):
            j = 3 * m + p
            process(p)
            r = (p + 2) % 3
            if p == 0:
                @pl.when(m > 0)
                def _():
                    drain_scatter(r)
            else:
                drain_scatter(r)

            @pl.when(j + 2 < K_CHUNKS)
            def _():
                issue(r, j + 2)
        return 0
    lax.fori_loop(0, K_CHUNKS // 3, ring, 0)
    drain_scatter(2)

    plsc.subcore_barrier()
    pltpu.sync_copy(acc_sh.at[pl.ds(tb, ROWS_PER_TILE)],
                    accout.at[c, pl.ds(tb, ROWS_PER_TILE)])
    pltpu.sync_copy(den_sh.at[pl.ds(tb, ROWS_PER_TILE)], den_st)
    pltpu.sync_copy(den_st, denout.at[pl.ds(c * NP + tb, ROWS_PER_TILE)])


def _edge_aggregate(xl, xr, edges, att):
    mesh = plsc.VectorSubcoreMesh(core_axis_name="c", subcore_axis_name="s")
    f = functools.partial(
        pl.kernel,
        mesh=mesh,
        compiler_params=pltpu.CompilerParams(needs_layout_passes=False),
        out_type=[
            jax.ShapeDtypeStruct((NC, NP, D), jnp.float32),
            jax.ShapeDtypeStruct((NC * NP,), jnp.float32),
        ],
        scratch_types=(
            [pltpu.VMEM((CH,), jnp.int32)] * 6
            + [pltpu.VMEM((CH, D), jnp.float32)] * 6
            + [pltpu.VMEM((D,), jnp.float32)]
            + [pltpu.VMEM((CH,), jnp.float32)] * 3
            + [pltpu.VMEM((ROWS_PER_TILE,), jnp.float32)]
            + [pltpu.VMEM_SHARED((NP, D), jnp.float32)]
            + [pltpu.VMEM_SHARED((NP,), jnp.float32)]
            + [pltpu.SemaphoreType.DMA] * 6
        ),
    )(_edge_body)
    return f(xl, xr, edges, att)


def _finish_body(a0_ref, a1_ref, d_ref, b_ref, bias_ref, out_ref,
                 pacc, cacc):
    i = pl.program_id(0)

    @pl.when(i == 0)
    def _():
        pacc[...] = jnp.zeros_like(pacc)
        cacc[...] = jnp.zeros_like(cacc)

    v = a0_ref[0] + a1_ref[0]
    d = jnp.sum(d_ref[...], axis=1, keepdims=True) + 1e-16
    u = v / d + bias_ref[...]
    u = jnp.where(u > 0, u, jnp.exp(jnp.minimum(u, 0.0)) - 1.0)

    bvec = b_ref[0]
    gid = lax.broadcasted_iota(jnp.int32, (G, bvec.shape[1]), 0)
    oh = (gid == bvec).astype(jnp.float32)
    pacc[...] += jnp.dot(oh, u, preferred_element_type=jnp.float32)
    cacc[...] += jnp.sum(oh, axis=1, keepdims=True)

    @pl.when(i == pl.num_programs(0) - 1)
    def _():
        out_ref[...] = pacc[...] / jnp.maximum(cacc[...], 1.0)


def _finish(acc, dent, batch_f, bias):
    bn = BN
    grid = (NP // bn,)
    return pl.pallas_call(
        _finish_body,
        grid=grid,
        in_specs=[
            pl.BlockSpec((1, bn, D), lambda i: (0, i, 0)),
            pl.BlockSpec((1, bn, D), lambda i: (1, i, 0)),
            pl.BlockSpec((bn, NC), lambda i: (i, 0)),
            pl.BlockSpec((1, 1, bn), lambda i: (i, 0, 0)),
            pl.BlockSpec((1, D), lambda i: (0, 0)),
        ],
        out_specs=pl.BlockSpec((G, D), lambda i: (0, 0)),
        out_shape=jax.ShapeDtypeStruct((G, D), jnp.float32),
        scratch_shapes=[
            pltpu.VMEM((G, D), jnp.float32),
            pltpu.VMEM((G, 1), jnp.float32),
        ],
    )(acc, acc, dent, batch_f, bias)


def kernel(x, edge_index, batch, W_l, W_r, att, bias):
    x_pad = jnp.pad(x, ((0, NP - N), (0, 0)))
    xl, xr = _node_transforms(x_pad, W_l, W_r)

    loop = jnp.arange(N, dtype=jnp.int32)
    pad_idx = (N + jnp.arange(EP - ET, dtype=jnp.int32) % (NP - N))
    edges = jnp.concatenate([edge_index[0], loop, pad_idx,
                             edge_index[1], loop, pad_idx])

    acc, den = _edge_aggregate(xl, xr, edges, att)

    batch_f = jnp.concatenate(
        [batch, jnp.full((NP - N,), G, jnp.int32)])
    pooled = _finish(acc, den.reshape(NC, NP).T,
                     batch_f.reshape(NP // BN, 1, BN), bias.reshape(1, D))
    return pooled


# speedup vs baseline: 1.4910x; 1.4910x over previous
import functools

import jax
import jax.numpy as jnp
from jax import lax
from jax.experimental import pallas as pl
from jax.experimental.pallas import tpu as pltpu
from jax.experimental.pallas import tpu_sc as plsc

N = 10000
E = 320000
D = 128
G = 64

NP = 10112
NC = 2
NS = 16
NW = NC * NS
CH = 48
ET = E + N
K_CHUNKS = 216
EP = NW * CH * K_CHUNKS
ROWS_PER_TILE = NP // NS
BN = 1264


def _mm_body(x_ref, wl_ref, wr_ref, xl_ref, xr_ref):
    x = x_ref[...]
    xl_ref[...] = jnp.dot(x, wl_ref[...], preferred_element_type=jnp.float32)
    xr_ref[...] = jnp.dot(x, wr_ref[...], preferred_element_type=jnp.float32)


def _node_transforms(x_pad, W_l, W_r):
    bn = BN
    grid = (NP // bn,)
    return pl.pallas_call(
        _mm_body,
        grid=grid,
        in_specs=[
            pl.BlockSpec((bn, D), lambda i: (i, 0)),
            pl.BlockSpec((D, D), lambda i: (0, 0)),
            pl.BlockSpec((D, D), lambda i: (0, 0)),
        ],
        out_specs=[
            pl.BlockSpec((bn, D), lambda i: (i, 0)),
            pl.BlockSpec((bn, D), lambda i: (i, 0)),
        ],
        out_shape=[
            jax.ShapeDtypeStruct((NP, D), jnp.float32),
            jax.ShapeDtypeStruct((NP, D), jnp.float32),
        ],
    )(x_pad, W_l, W_r)


def _edge_body(xl, xr, edges, atth, accout, denout,
               src0, dst0, src1, dst1, src2, dst2,
               A0, B0, A1, B1, A2, B2,
               att_v, exb0, exb1, exb2, den_st, acc_sh, den_sh,
               s0, s1, s2, t0, t1, t2):
    c = lax.axis_index("c")
    s = lax.axis_index("s")
    wid = c * NS + s
    tb = s * ROWS_PER_TILE
    zero16 = jnp.zeros((16,), jnp.float32)
    lane = lax.iota(jnp.int32, 16)
    m0 = lane == 0

    A = A0

    def zrow(i, _):
        for v in range(8):
            A[i, pl.ds(v * 16, 16)] = zero16
        return 0
    lax.fori_loop(0, CH, zrow, 0)

    for v in range(CH // 16):
        exb0[pl.ds(v * 16, 16)] = zero16
    for j in range(ROWS_PER_TILE // CH):
        pltpu.sync_copy(A, acc_sh.at[pl.ds(tb + j * CH, CH)])
    tail = ROWS_PER_TILE % CH
    if tail:
        tb2 = tb + (ROWS_PER_TILE // CH) * CH
        pltpu.sync_copy(A.at[pl.ds(0, tail)], acc_sh.at[pl.ds(tb2, tail)])
    for j in range(ROWS_PER_TILE // CH):
        pltpu.sync_copy(exb0, den_sh.at[pl.ds(tb + j * CH, CH)])
    if tail:
        tb2 = tb + (ROWS_PER_TILE // CH) * CH
        pltpu.sync_copy(exb0.at[pl.ds(0, tail)], den_sh.at[pl.ds(tb2, tail)])
    pltpu.sync_copy(atth, att_v)
    plsc.subcore_barrier()

    att8 = [att_v[pl.ds(v * 16, 16)] for v in range(8)]
    ebase = wid * (K_CHUNKS * CH)
    bufs = ((src0, dst0, A0, B0, exb0, s0, t0),
            (src1, dst1, A1, B1, exb1, s1, t1),
            (src2, dst2, A2, B2, exb2, s2, t2))

    def issue(p, j):
        sv, dv, Ab, Bb, eb, sm, st = bufs[p]
        base = ebase + j * CH
        pltpu.sync_copy(edges.at[pl.ds(base, CH)], sv)
        pltpu.sync_copy(edges.at[pl.ds(EP + base, CH)], dv)
        pltpu.async_copy(xl.at[sv], Ab, sm)
        pltpu.async_copy(xr.at[dv], Bb, sm)

    def drain_scatter(p):
        sv, dv, Ab, Bb, eb, sm, st = bufs[p]
        pltpu.make_async_copy(Ab, acc_sh.at[dv], st).wait()
        pltpu.make_async_copy(eb, den_sh.at[dv], st).wait()

    def process(p):
        sv, dv, Ab, Bb, eb, sm, st = bufs[p]
        pltpu.make_async_copy(xl.at[sv], Ab, sm).wait()
        pltpu.make_async_copy(xr.at[dv], Bb, sm).wait()

        @plsc.parallel_loop(0, CH, step=4, unroll=4)
        def ibody(t):
            edges = tuple(t + u for u in range(4))
            avss = []
            evs = []
            for i in edges:
                avs = []
                pv = zero16
                ms = []
                for v in range(8):
                    a = Ab[i, pl.ds(v * 16, 16)]
                    b = Bb[i, pl.ds(v * 16, 16)]
                    avs.append(a)
                    z = a + b
                    z = jnp.maximum(z, z * 0.2)
                    ms.append(z * att8[v])
                pv = (((ms[0] + ms[1]) + (ms[2] + ms[3]))
                      + ((ms[4] + ms[5]) + (ms[6] + ms[7])))
                avss.append(avs)
                evs.append(jnp.exp(jnp.full((16,), jnp.sum(pv),
                                            jnp.float32)))
            for n, i in enumerate(edges):
                for v in range(8):
                    Ab[i, pl.ds(v * 16, 16)] = avss[n][v] * evs[n]
                plsc.store_scatter(eb, [jnp.full((16,), i, jnp.int32)],
                                   evs[n], mask=m0)

        pltpu.async_copy(Ab, acc_sh.at[dv], st, add=True)
        pltpu.async_copy(eb, den_sh.at[dv], st, add=True)

    issue(0, 0)
    issue(1, 1)

    def ring(m, _):
        for p in range(3):
            j = 3 * m + p
            process(p)
            r = (p + 2) % 3
            if p == 0:
                @pl.when(m > 0)
                def _():
                    drain_scatter(r)
            else:
                drain_scatter(r)

            @pl.when(j + 2 < K_CHUNKS)
            def _():
                issue(r, j + 2)
        return 0
    lax.fori_loop(0, K_CHUNKS // 3, ring, 0)
    drain_scatter(2)

    plsc.subcore_barrier()
    pltpu.sync_copy(acc_sh.at[pl.ds(tb, ROWS_PER_TILE)],
                    accout.at[c, pl.ds(tb, ROWS_PER_TILE)])
    pltpu.sync_copy(den_sh.at[pl.ds(tb, ROWS_PER_TILE)], den_st)
    pltpu.sync_copy(den_st, denout.at[pl.ds(c * NP + tb, ROWS_PER_TILE)])


def _edge_aggregate(xl, xr, edges, att):
    mesh = plsc.VectorSubcoreMesh(core_axis_name="c", subcore_axis_name="s")
    f = functools.partial(
        pl.kernel,
        mesh=mesh,
        compiler_params=pltpu.CompilerParams(needs_layout_passes=False),
        out_type=[
            jax.ShapeDtypeStruct((NC, NP, D), jnp.float32),
            jax.ShapeDtypeStruct((NC * NP,), jnp.float32),
        ],
        scratch_types=(
            [pltpu.VMEM((CH,), jnp.int32)] * 6
            + [pltpu.VMEM((CH, D), jnp.float32)] * 6
            + [pltpu.VMEM((D,), jnp.float32)]
            + [pltpu.VMEM((CH,), jnp.float32)] * 3
            + [pltpu.VMEM((ROWS_PER_TILE,), jnp.float32)]
            + [pltpu.VMEM_SHARED((NP, D), jnp.float32)]
            + [pltpu.VMEM_SHARED((NP,), jnp.float32)]
            + [pltpu.SemaphoreType.DMA] * 6
        ),
    )(_edge_body)
    return f(xl, xr, edges, att)


def _finish_body(a0_ref, a1_ref, d_ref, b_ref, bias_ref, out_ref,
                 pacc, cacc):
    i = pl.program_id(0)

    @pl.when(i == 0)
    def _():
        pacc[...] = jnp.zeros_like(pacc)
        cacc[...] = jnp.zeros_like(cacc)

    v = a0_ref[0] + a1_ref[0]
    d = jnp.sum(d_ref[...], axis=1, keepdims=True) + 1e-16
    u = v / d + bias_ref[...]
    u = jnp.where(u > 0, u, jnp.exp(jnp.minimum(u, 0.0)) - 1.0)

    bvec = b_ref[0]
    gid = lax.broadcasted_iota(jnp.int32, (G, bvec.shape[1]), 0)
    oh = (gid == bvec).astype(jnp.float32)
    pacc[...] += jnp.dot(oh, u, preferred_element_type=jnp.float32)
    cacc[...] += jnp.sum(oh, axis=1, keepdims=True)

    @pl.when(i == pl.num_programs(0) - 1)
    def _():
        out_ref[...] = pacc[...] / jnp.maximum(cacc[...], 1.0)


def _finish(acc, dent, batch_f, bias):
    bn = BN
    grid = (NP // bn,)
    return pl.pallas_call(
        _finish_body,
        grid=grid,
        in_specs=[
            pl.BlockSpec((1, bn, D), lambda i: (0, i, 0)),
            pl.BlockSpec((1, bn, D), lambda i: (1, i, 0)),
            pl.BlockSpec((bn, NC), lambda i: (i, 0)),
            pl.BlockSpec((1, 1, bn), lambda i: (i, 0, 0)),
            pl.BlockSpec((1, D), lambda i: (0, 0)),
        ],
        out_specs=pl.BlockSpec((G, D), lambda i: (0, 0)),
        out_shape=jax.ShapeDtypeStruct((G, D), jnp.float32),
        scratch_shapes=[
            pltpu.VMEM((G, D), jnp.float32),
            pltpu.VMEM((G, 1), jnp.float32),
        ],
    )(acc, acc, dent, batch_f, bias)


def kernel(x, edge_index, batch, W_l, W_r, att, bias):
    x_pad = jnp.pad(x, ((0, NP - N), (0, 0)))
    xl, xr = _node_transforms(x_pad, W_l, W_r)

    loop = jnp.arange(N, dtype=jnp.int32)
    pad_idx = (N + jnp.arange(EP - ET, dtype=jnp.int32) % (NP - N))
    edges = jnp.concatenate([edge_index[0], loop, pad_idx,
                             edge_index[1], loop, pad_idx])

    acc, den = _edge_aggregate(xl, xr, edges, att)

    batch_f = jnp.concatenate(
        [batch, jnp.full((NP - N,), G, jnp.int32)])
    pooled = _finish(acc, den.reshape(NC, NP).T,
                     batch_f.reshape(NP // BN, 1, BN), bias.reshape(1, D))
    return pooled
